# baseline (speedup 1.0000x reference)
import functools

import jax
import jax.numpy as jnp
from jax import lax
from jax.experimental import pallas as pl
from jax.experimental.pallas import tpu as pltpu
from jax.experimental.pallas import tpu_sc as plsc

_N_CHUNKS = 4


def _matmul_body(emb_ref, w_ref, b_ref, out_ref):
    out_ref[...] = (
        jnp.dot(emb_ref[...], w_ref[...], preferred_element_type=jnp.float32)
        + b_ref[...]
    )


@functools.partial(jax.jit, static_argnames=("bs", "d", "he", "t"))
def _all_bank_projection(emb2d, w2d, b2d, *, bs, d, he, t):
    grid = (bs // t,)
    return pl.pallas_call(
        _matmul_body,
        grid=grid,
        in_specs=[
            pl.BlockSpec((t, d), lambda i: (i, 0)),
            pl.BlockSpec((d, he), lambda i: (0, 0)),
            pl.BlockSpec((1, he), lambda i: (0, 0)),
        ],
        out_specs=pl.BlockSpec((t, he), lambda i: (i, 0)),
        out_shape=jax.ShapeDtypeStruct((bs, he), jnp.float32),
    )(emb2d, w2d, b2d)


def _make_sc_gather(rows, e, nc, ns):
    nw = nc * ns
    per_w = rows // nw
    chunk = 128
    n_chunks = per_w // chunk
    mesh = plsc.VectorSubcoreMesh(core_axis_name="c", subcore_axis_name="s")

    @functools.partial(
        pl.kernel,
        out_type=jax.ShapeDtypeStruct((rows, e), jnp.float32),
        mesh=mesh,
        compiler_params=pltpu.CompilerParams(use_tc_tiling_on_sc=False),
        scratch_types=[
            pltpu.VMEM((chunk,), jnp.int32),
            pltpu.VMEM((chunk, e), jnp.float32),
            pltpu.SemaphoreType.DMA,
        ],
    )
    def gather_kernel(table_hbm, idx_hbm, out_hbm, idx_v, rows_v, sem):
        wid = lax.axis_index("s") * nc + lax.axis_index("c")
        base = wid * per_w

        def do_chunk(c, carry):
            start = base + c * chunk
            pltpu.sync_copy(idx_hbm.at[pl.ds(start, chunk)], idx_v)
            pltpu.async_copy(table_hbm.at[idx_v], rows_v, sem).wait()
            pltpu.sync_copy(rows_v, out_hbm.at[pl.ds(start, chunk)])
            return carry

        lax.fori_loop(0, n_chunks, do_chunk, 0)

    return gather_kernel


def kernel(embedding, selection_idx, selection_prob, W, b):
    del selection_prob
    bb, s, d = embedding.shape
    h, _, e = W.shape
    k = selection_idx.shape[-1]
    bs = bb * s

    emb2d = embedding.reshape(bs, d)
    w2d = jnp.transpose(W, (1, 0, 2)).reshape(d, h * e)
    b2d = b.reshape(1, h * e)
    sel_flat = selection_idx.astype(jnp.int32).reshape(bs * k)

    cbs = bs // _N_CHUNKS
    tok_ids = jnp.arange(cbs * k, dtype=jnp.int32) // k

    info = plsc.get_sparse_core_info()
    sc_gather = _make_sc_gather(cbs * k, e, info.num_cores, info.num_subcores)

    outs = []
    for c in range(_N_CHUNKS):
        emb_c = lax.slice_in_dim(emb2d, c * cbs, (c + 1) * cbs, axis=0)
        all_out = _all_bank_projection(emb_c, w2d, b2d, bs=cbs, d=d, he=h * e, t=512)
        table = all_out.reshape(cbs * h, e)
        sel_c = lax.slice_in_dim(sel_flat, c * cbs * k, (c + 1) * cbs * k, axis=0)
        flat_idx = tok_ids * h + sel_c
        outs.append(sc_gather(table, flat_idx))

    gathered = jnp.concatenate(outs, axis=0)
    return gathered.reshape(bb, s, k, e)


# speedup vs baseline: 1.5134x; 1.5134x over previous
import functools

import jax
import jax.numpy as jnp
from jax import lax
from jax.experimental import pallas as pl
from jax.experimental.pallas import tpu as pltpu
from jax.experimental.pallas import tpu_sc as plsc


def _matmul_body(emb_ref, w_ref, b_ref, out_ref):
    out_ref[...] = (
        jnp.dot(emb_ref[...], w_ref[...], preferred_element_type=jnp.float32)
        + b_ref[...]
    )


@functools.partial(jax.jit, static_argnames=("bs", "d", "he", "t"))
def _all_bank_projection(emb2d, w2d, b2d, *, bs, d, he, t):
    grid = (bs // t,)
    return pl.pallas_call(
        _matmul_body,
        grid=grid,
        in_specs=[
            pl.BlockSpec((t, d), lambda i: (i, 0)),
            pl.BlockSpec((d, he), lambda i: (0, 0)),
            pl.BlockSpec((1, he), lambda i: (0, 0)),
        ],
        out_specs=pl.BlockSpec((t, he), lambda i: (i, 0)),
        out_shape=jax.ShapeDtypeStruct((bs, he), jnp.float32),
    )(emb2d, w2d, b2d)


def _make_sc_gather(rows, e, nc, ns):
    nw = nc * ns
    per_w = rows // nw
    chunk = 128
    n_chunks = per_w // chunk
    nbuf = min(4, n_chunks)
    mesh = plsc.VectorSubcoreMesh(core_axis_name="c", subcore_axis_name="s")

    scratch_types = (
        [pltpu.VMEM((per_w,), jnp.int32)]
        + [pltpu.VMEM((chunk, e), jnp.float32) for _ in range(nbuf)]
        + [pltpu.SemaphoreType.DMA for _ in range(nbuf)]
    )

    @functools.partial(
        pl.kernel,
        out_type=jax.ShapeDtypeStruct((rows, e), jnp.float32),
        mesh=mesh,
        compiler_params=pltpu.CompilerParams(use_tc_tiling_on_sc=False),
        scratch_types=scratch_types,
    )
    def gather_kernel(table_hbm, idx_hbm, out_hbm, idx_all, *bufs_sems):
        bufs = bufs_sems[:nbuf]
        sems = bufs_sems[nbuf:]
        wid = lax.axis_index("s") * nc + lax.axis_index("c")
        base = wid * per_w

        pltpu.sync_copy(idx_hbm.at[pl.ds(base, per_w)], idx_all)

        def issue(c):
            return pltpu.async_copy(
                table_hbm.at[idx_all.at[pl.ds(c * chunk, chunk)]],
                bufs[c % nbuf],
                sems[c % nbuf],
            )

        handles = [None] * n_chunks
        for c in range(nbuf):
            handles[c] = issue(c)
        for c in range(n_chunks):
            handles[c].wait()
            pltpu.sync_copy(bufs[c % nbuf], out_hbm.at[pl.ds(base + c * chunk, chunk)])
            if c + nbuf < n_chunks:
                handles[c + nbuf] = issue(c + nbuf)

    return gather_kernel


def kernel(embedding, selection_idx, selection_prob, W, b):
    del selection_prob
    bb, s, d = embedding.shape
    h, _, e = W.shape
    k = selection_idx.shape[-1]
    bs = bb * s

    emb2d = embedding.reshape(bs, d)
    w2d = jnp.transpose(W, (1, 0, 2)).reshape(d, h * e)
    b2d = b.reshape(1, h * e)
    all_out = _all_bank_projection(emb2d, w2d, b2d, bs=bs, d=d, he=h * e, t=512)

    table = all_out.reshape(bs * h, e)
    sel_flat = selection_idx.astype(jnp.int32).reshape(bs * k)
    tok_ids = jnp.arange(bs * k, dtype=jnp.int32) // k
    flat_idx = tok_ids * h + sel_flat

    info = plsc.get_sparse_core_info()
    gathered = _make_sc_gather(bs * k, e, info.num_cores, info.num_subcores)(
        table, flat_idx
    )
    return gathered.reshape(bb, s, k, e)


# speedup vs baseline: 1.5145x; 1.0007x over previous
import functools

import jax
import jax.numpy as jnp
from jax import lax
from jax.experimental import pallas as pl
from jax.experimental.pallas import tpu as pltpu
from jax.experimental.pallas import tpu_sc as plsc


def _matmul_body(emb_ref, w_ref, b_ref, out_ref):
    out_ref[...] = (
        jnp.dot(
            emb_ref[...].astype(jnp.bfloat16),
            w_ref[...].astype(jnp.bfloat16),
            preferred_element_type=jnp.float32,
        )
        + b_ref[...]
    )


@functools.partial(jax.jit, static_argnames=("bs", "d", "he", "t"))
def _all_bank_projection(emb2d, w2d, b2d, *, bs, d, he, t):
    grid = (bs // t,)
    return pl.pallas_call(
        _matmul_body,
        grid=grid,
        in_specs=[
            pl.BlockSpec((t, d), lambda i: (i, 0)),
            pl.BlockSpec((d, he), lambda i: (0, 0)),
            pl.BlockSpec((1, he), lambda i: (0, 0)),
        ],
        out_specs=pl.BlockSpec((t, he), lambda i: (i, 0)),
        out_shape=jax.ShapeDtypeStruct((bs, he), jnp.float32),
    )(emb2d, w2d, b2d)


def _make_sc_gather(rows, e, nc, ns):
    nw = nc * ns
    per_w = rows // nw
    chunk = 128
    n_chunks = per_w // chunk
    nbuf = min(4, n_chunks)
    mesh = plsc.VectorSubcoreMesh(core_axis_name="c", subcore_axis_name="s")

    scratch_types = (
        [pltpu.VMEM((per_w,), jnp.int32)]
        + [pltpu.VMEM((chunk, e), jnp.float32) for _ in range(nbuf)]
        + [pltpu.SemaphoreType.DMA for _ in range(nbuf)]
    )

    @functools.partial(
        pl.kernel,
        out_type=jax.ShapeDtypeStruct((rows, e), jnp.float32),
        mesh=mesh,
        compiler_params=pltpu.CompilerParams(use_tc_tiling_on_sc=False),
        scratch_types=scratch_types,
    )
    def gather_kernel(table_hbm, idx_hbm, out_hbm, idx_all, *bufs_sems):
        bufs = bufs_sems[:nbuf]
        sems = bufs_sems[nbuf:]
        wid = lax.axis_index("s") * nc + lax.axis_index("c")
        base = wid * per_w

        pltpu.sync_copy(idx_hbm.at[pl.ds(base, per_w)], idx_all)

        def issue(c):
            return pltpu.async_copy(
                table_hbm.at[idx_all.at[pl.ds(c * chunk, chunk)]],
                bufs[c % nbuf],
                sems[c % nbuf],
            )

        handles = [None] * n_chunks
        for c in range(nbuf):
            handles[c] = issue(c)
        for c in range(n_chunks):
            handles[c].wait()
            pltpu.sync_copy(bufs[c % nbuf], out_hbm.at[pl.ds(base + c * chunk, chunk)])
            if c + nbuf < n_chunks:
                handles[c + nbuf] = issue(c + nbuf)

    return gather_kernel


def kernel(embedding, selection_idx, selection_prob, W, b):
    del selection_prob
    bb, s, d = embedding.shape
    h, _, e = W.shape
    k = selection_idx.shape[-1]
    bs = bb * s

    emb2d = embedding.reshape(bs, d)
    w2d = jnp.transpose(W, (1, 0, 2)).reshape(d, h * e)
    b2d = b.reshape(1, h * e)
    all_out = _all_bank_projection(emb2d, w2d, b2d, bs=bs, d=d, he=h * e, t=512)

    table = all_out.reshape(bs * h, e)
    sel_flat = selection_idx.astype(jnp.int32).reshape(bs * k)
    tok_ids = jnp.arange(bs * k, dtype=jnp.int32) // k
    flat_idx = tok_ids * h + sel_flat

    info = plsc.get_sparse_core_info()
    gathered = _make_sc_gather(bs * k, e, info.num_cores, info.num_subcores)(
        table, flat_idx
    )
    return gathered.reshape(bb, s, k, e)


# speedup vs baseline: 1.7539x; 1.1581x over previous
import functools

import jax
import jax.numpy as jnp
from jax import lax
from jax.experimental import pallas as pl
from jax.experimental.pallas import tpu as pltpu
from jax.experimental.pallas import tpu_sc as plsc


def _matmul_body(emb_ref, w_ref, b_ref, out_ref):
    res = (
        jnp.dot(
            emb_ref[...].astype(jnp.bfloat16),
            w_ref[...].astype(jnp.bfloat16),
            preferred_element_type=jnp.float32,
        )
        + b_ref[...]
    )
    out_ref[...] = res.reshape(res.shape[0], 8, 128)


@functools.partial(jax.jit, static_argnames=("bs", "d", "he", "t"))
def _all_bank_projection(emb2d, w2d, b2d, *, bs, d, he, t):
    grid = (bs // t,)
    return pl.pallas_call(
        _matmul_body,
        grid=grid,
        in_specs=[
            pl.BlockSpec((t, d), lambda i: (i, 0)),
            pl.BlockSpec((d, he), lambda i: (0, 0)),
            pl.BlockSpec((1, he), lambda i: (0, 0)),
        ],
        out_specs=pl.BlockSpec((t, he // 128, 128), lambda i: (i, 0, 0)),
        out_shape=jax.ShapeDtypeStruct((bs, he // 128, 128), jnp.float32),
    )(emb2d, w2d, b2d)


def _make_sc_gather(rows, e, nc, ns):
    nw = nc * ns
    per_w = rows // nw
    chunk = 128
    n_chunks = per_w // chunk
    nbuf = min(4, n_chunks)
    mesh = plsc.VectorSubcoreMesh(core_axis_name="c", subcore_axis_name="s")

    scratch_types = (
        [pltpu.VMEM((per_w,), jnp.int32)]
        + [pltpu.VMEM((chunk, e), jnp.float32) for _ in range(nbuf)]
        + [pltpu.SemaphoreType.DMA for _ in range(nbuf)]
    )

    @functools.partial(
        pl.kernel,
        out_type=jax.ShapeDtypeStruct((rows, e), jnp.float32),
        mesh=mesh,
        compiler_params=pltpu.CompilerParams(use_tc_tiling_on_sc=False),
        scratch_types=scratch_types,
    )
    def gather_kernel(table_hbm, idx_hbm, out_hbm, idx_all, *bufs_sems):
        bufs = bufs_sems[:nbuf]
        sems = bufs_sems[nbuf:]
        wid = lax.axis_index("s") * nc + lax.axis_index("c")
        base = wid * per_w

        pltpu.sync_copy(idx_hbm.at[pl.ds(base, per_w)], idx_all)

        def issue(c):
            return pltpu.async_copy(
                table_hbm.at[idx_all.at[pl.ds(c * chunk, chunk)]],
                bufs[c % nbuf],
                sems[c % nbuf],
            )

        handles = [None] * n_chunks
        for c in range(nbuf):
            handles[c] = issue(c)
        for c in range(n_chunks):
            handles[c].wait()
            pltpu.sync_copy(bufs[c % nbuf], out_hbm.at[pl.ds(base + c * chunk, chunk)])
            if c + nbuf < n_chunks:
                handles[c + nbuf] = issue(c + nbuf)

    return gather_kernel


def kernel(embedding, selection_idx, selection_prob, W, b):
    del selection_prob
    bb, s, d = embedding.shape
    h, _, e = W.shape
    k = selection_idx.shape[-1]
    bs = bb * s

    emb2d = embedding.reshape(bs, d)
    w2d = jnp.transpose(W, (1, 0, 2)).reshape(d, h * e)
    b2d = b.reshape(1, h * e)
    all_out = _all_bank_projection(emb2d, w2d, b2d, bs=bs, d=d, he=h * e, t=512)

    table = all_out.reshape(bs * h, e)
    sel_flat = selection_idx.astype(jnp.int32).reshape(bs * k)
    tok_ids = jnp.arange(bs * k, dtype=jnp.int32) // k
    flat_idx = tok_ids * h + sel_flat

    info = plsc.get_sparse_core_info()
    gathered = _make_sc_gather(bs * k, e, info.num_cores, info.num_subcores)(
        table, flat_idx
    )
    return gathered.reshape(bb, s, k, e)


# speedup vs baseline: 1.9603x; 1.1177x over previous
import functools

import jax
import jax.numpy as jnp
from jax import lax
from jax.experimental import pallas as pl
from jax.experimental.pallas import tpu as pltpu
from jax.experimental.pallas import tpu_sc as plsc


def _matmul_body(emb_ref, w_ref, b_ref, out_ref, w_s):
    @pl.when(pl.program_id(0) == 0)
    def _():
        h = w_ref.shape[0]
        e = w_ref.shape[2]
        for j in range(h):
            w_s[:, j * e : (j + 1) * e] = w_ref[j].astype(jnp.bfloat16)

    res = jnp.dot(
        emb_ref[...].astype(jnp.bfloat16),
        w_s[...],
        preferred_element_type=jnp.float32,
    )
    out_ref[...] = res.reshape(res.shape[0], 8, 128) + b_ref[...]


@functools.partial(jax.jit, static_argnames=("bs", "d", "he", "t"))
def _all_bank_projection(emb2d, w3, b2, *, bs, d, he, t):
    grid = (bs // t,)
    h, _, e = w3.shape
    return pl.pallas_call(
        _matmul_body,
        grid=grid,
        in_specs=[
            pl.BlockSpec((t, d), lambda i: (i, 0)),
            pl.BlockSpec((h, d, e), lambda i: (0, 0, 0)),
            pl.BlockSpec((1, he // 128, 128), lambda i: (0, 0, 0)),
        ],
        out_specs=pl.BlockSpec((t, he // 128, 128), lambda i: (i, 0, 0)),
        out_shape=jax.ShapeDtypeStruct((bs, he // 128, 128), jnp.float32),
        scratch_shapes=[pltpu.VMEM((d, he), jnp.bfloat16)],
    )(emb2d, w3, b2)


def _make_sc_gather(rows, e, nc, ns):
    nw = nc * ns
    per_w = rows // nw
    chunk = 128
    n_chunks = per_w // chunk
    nbuf = min(4, n_chunks)
    mesh = plsc.VectorSubcoreMesh(core_axis_name="c", subcore_axis_name="s")

    scratch_types = (
        [pltpu.VMEM((per_w,), jnp.int32)]
        + [pltpu.VMEM((chunk, e), jnp.float32) for _ in range(nbuf)]
        + [pltpu.SemaphoreType.DMA for _ in range(nbuf)]
    )

    @functools.partial(
        pl.kernel,
        out_type=jax.ShapeDtypeStruct((rows, 128), jnp.float32),
        mesh=mesh,
        compiler_params=pltpu.CompilerParams(use_tc_tiling_on_sc=False),
        scratch_types=scratch_types,
    )
    def gather_kernel(table_hbm, idx_hbm, out_hbm, idx_all, *bufs_sems):
        bufs = bufs_sems[:nbuf]
        sems = bufs_sems[nbuf:]
        wid = lax.axis_index("s") * nc + lax.axis_index("c")
        base = wid * per_w

        pltpu.sync_copy(idx_hbm.at[pl.ds(base, per_w)], idx_all)

        def issue(c):
            return pltpu.async_copy(
                table_hbm.at[idx_all.at[pl.ds(c * chunk, chunk)]],
                bufs[c % nbuf],
                sems[c % nbuf],
            )

        handles = [None] * n_chunks
        for c in range(nbuf):
            handles[c] = issue(c)
        for c in range(n_chunks):
            handles[c].wait()
            pltpu.sync_copy(
                bufs[c % nbuf],
                out_hbm.at[pl.ds(base + c * chunk, chunk), pl.ds(0, e)],
            )
            if c + nbuf < n_chunks:
                handles[c + nbuf] = issue(c + nbuf)

    return gather_kernel


def kernel(embedding, selection_idx, selection_prob, W, b):
    del selection_prob
    bb, s, d = embedding.shape
    h, _, e = W.shape
    k = selection_idx.shape[-1]
    bs = bb * s

    emb2d = embedding.reshape(bs, d)
    b2 = b.reshape(1, h * e // 128, 128)
    all_out = _all_bank_projection(emb2d, W, b2, bs=bs, d=d, he=h * e, t=512)

    table = all_out.reshape(bs * h, e)
    sel_flat = selection_idx.astype(jnp.int32).reshape(bs * k)
    tok_ids = jnp.arange(bs * k, dtype=jnp.int32) // k
    flat_idx = tok_ids * h + sel_flat

    info = plsc.get_sparse_core_info()
    gathered = _make_sc_gather(bs * k, e, info.num_cores, info.num_subcores)(
        table, flat_idx
    )
    return gathered[:, :e].reshape(bb, s, k, e)
